# baseline (speedup 1.0000x reference)
import functools

import jax
import jax.numpy as jnp
from jax import lax
from jax.experimental import pallas as pl
from jax.experimental.pallas import tpu as pltpu
from jax.experimental.pallas import tpu_sc as plsc

VOCAB = 1000000
SEQ = 200
BATCH = 4096
DIM = 64

NUM_CORES = 2
NUM_SUBCORES = 16
NW = NUM_CORES * NUM_SUBCORES
ROWS = BATCH * SEQ
ROWS_PER_W = ROWS // NW
CHUNK = 2 * SEQ
NCHUNK = ROWS_PER_W // CHUNK

_Q = DIM // 16


def _rsqrt16(v):
    i = lax.bitcast_convert_type(v, jnp.int32)
    i = jnp.int32(0x5F3759DF) - lax.shift_right_logical(i, 1)
    y = lax.bitcast_convert_type(i, jnp.float32)
    half = v * 0.5
    for _ in range(3):
        y = y * (1.5 - half * y * y)
    return y


def _sc_body(ids_hbm, table_hbm, pos_hbm, gamma_hbm, beta_hbm, out_hbm,
             idx_v, buf_v, pos_v, gb_v, sem):
    wid = lax.axis_index("s") * NUM_CORES + lax.axis_index("c")
    w_base = wid * ROWS_PER_W

    pltpu.sync_copy(pos_hbm, pos_v)
    pltpu.sync_copy(gamma_hbm, gb_v.at[0])
    pltpu.sync_copy(beta_hbm, gb_v.at[1])

    def chunk_body(ci, _):
        base = w_base + ci * CHUNK
        pltpu.sync_copy(ids_hbm.at[pl.ds(base, CHUNK)], idx_v)
        pltpu.async_copy(table_hbm.at[idx_v], buf_v, sem).wait()

        def row_body(r, _):
            x = [buf_v[r, pl.ds(16 * q, 16)] + pos_v[r, pl.ds(16 * q, 16)]
                 for q in range(_Q)]
            s = jnp.sum(x[0] + x[1] + x[2] + x[3])
            ss = jnp.sum(x[0] * x[0] + x[1] * x[1]
                         + x[2] * x[2] + x[3] * x[3])
            mean = s * (1.0 / DIM)
            var = ss * (1.0 / DIM) - mean * mean
            varv = jnp.broadcast_to(var + 1e-12, (16,))
            rstd = _rsqrt16(varv)
            meanv = jnp.broadcast_to(mean, (16,))
            for q in range(_Q):
                g = gb_v[0, pl.ds(16 * q, 16)]
                b = gb_v[1, pl.ds(16 * q, 16)]
                buf_v[r, pl.ds(16 * q, 16)] = (x[q] - meanv) * rstd * g + b
            return 0

        lax.fori_loop(0, CHUNK, row_body, 0, unroll=False)
        pltpu.sync_copy(buf_v, out_hbm.at[pl.ds(base, CHUNK)])
        return 0

    lax.fori_loop(0, NCHUNK, chunk_body, 0, unroll=False)


@jax.jit
def _run(ids_flat, token_table, pos_tiled, gamma, beta):
    mesh = plsc.VectorSubcoreMesh(core_axis_name="c", subcore_axis_name="s")
    kern = functools.partial(
        pl.kernel,
        out_type=jax.ShapeDtypeStruct((ROWS, DIM), jnp.float32),
        mesh=mesh,
        compiler_params=pltpu.CompilerParams(
            needs_layout_passes=False, use_tc_tiling_on_sc=False),
        scratch_types=[
            pltpu.VMEM((CHUNK,), jnp.int32),
            pltpu.VMEM((CHUNK, DIM), jnp.float32),
            pltpu.VMEM((CHUNK, DIM), jnp.float32),
            pltpu.VMEM((2, DIM), jnp.float32),
            pltpu.SemaphoreType.DMA,
        ],
    )(_sc_body)
    return kern(ids_flat, token_table, pos_tiled, gamma, beta)


def kernel(input_ids, token_table, pos_table, gamma, beta):
    ids_flat = input_ids.reshape(-1).astype(jnp.int32)
    pos_tiled = jnp.concatenate([pos_table] * (CHUNK // SEQ), axis=0)
    out = _run(ids_flat, token_table, pos_tiled,
               gamma.astype(jnp.float32), beta.astype(jnp.float32))
    return out.reshape(BATCH, SEQ, DIM)


# speedup vs baseline: 1.1656x; 1.1656x over previous
import functools

import jax
import jax.numpy as jnp
from jax import lax
from jax.experimental import pallas as pl
from jax.experimental.pallas import tpu as pltpu
from jax.experimental.pallas import tpu_sc as plsc

VOCAB = 1000000
SEQ = 200
BATCH = 4096
DIM = 64

NUM_CORES = 2
NUM_SUBCORES = 16
NW = NUM_CORES * NUM_SUBCORES
ROWS = BATCH * SEQ
ROWS_PER_W = ROWS // NW
CHUNK = 2 * SEQ
NCHUNK = ROWS_PER_W // CHUNK
GROUP = 8
NGROUP = CHUNK // GROUP

_Q = DIM // 16


def _sc_body(ids_hbm, table_hbm, pos_hbm, gamma_hbm, beta_hbm, out_hbm,
             idx_v, buf_v, pos_v, gb_v, gsem, osem):
    wid = lax.axis_index("s") * NUM_CORES + lax.axis_index("c")
    w_base = wid * ROWS_PER_W

    pltpu.sync_copy(pos_hbm, pos_v)
    pltpu.sync_copy(gamma_hbm, gb_v.at[0])
    pltpu.sync_copy(beta_hbm, gb_v.at[1])

    pltpu.sync_copy(ids_hbm.at[pl.ds(w_base, CHUNK)], idx_v.at[0])
    pltpu.async_copy(table_hbm.at[idx_v.at[0]], buf_v.at[0], gsem.at[0])

    def compute_chunk(p):
        g = [gb_v[0, pl.ds(16 * q, 16)] for q in range(_Q)]
        b = [gb_v[1, pl.ds(16 * q, 16)] for q in range(_Q)]

        def group_body(gi, carry):
            r0 = gi * GROUP
            for j in range(GROUP):
                r = r0 + j
                x = [buf_v[p, r, pl.ds(16 * q, 16)]
                     + pos_v[r, pl.ds(16 * q, 16)] for q in range(_Q)]
                s = jnp.sum(x[0] + x[1] + x[2] + x[3])
                ss = jnp.sum(x[0] * x[0] + x[1] * x[1]
                             + x[2] * x[2] + x[3] * x[3])
                mean = s * (1.0 / DIM)
                var = ss * (1.0 / DIM) - mean * mean + 1e-12
                i32 = lax.bitcast_convert_type(var, jnp.int32)
                i32 = jnp.int32(0x5F3759DF) - lax.shift_right_logical(i32, 1)
                y = lax.bitcast_convert_type(i32, jnp.float32)
                half = var * 0.5
                y = y * (1.5 - half * y * y)
                y = y * (1.5 - half * y * y)
                y = y * (1.5 - half * y * y)
                meanv = jnp.broadcast_to(mean, (16,))
                rstdv = jnp.broadcast_to(y, (16,))
                for q in range(_Q):
                    buf_v[p, r, pl.ds(16 * q, 16)] = (
                        (x[q] - meanv) * (rstdv * g[q]) + b[q])
            return carry

        lax.fori_loop(0, NGROUP, group_body, 0, unroll=False)

    def chunk_body(ci, _):
        par = lax.rem(ci, 2)
        nxt = lax.rem(ci + 1, 2)

        @pl.when(ci + 1 < NCHUNK)
        def _prefetch():
            @pl.when(ci >= 1)
            def _drain_out():
                pltpu.make_async_copy(
                    buf_v.at[nxt],
                    out_hbm.at[pl.ds(w_base + (ci - 1) * CHUNK, CHUNK)],
                    osem.at[nxt]).wait()
            base_n = w_base + (ci + 1) * CHUNK
            pltpu.sync_copy(ids_hbm.at[pl.ds(base_n, CHUNK)], idx_v.at[nxt])
            pltpu.async_copy(table_hbm.at[idx_v.at[nxt]], buf_v.at[nxt],
                             gsem.at[nxt])

        pltpu.make_async_copy(table_hbm.at[idx_v.at[par]], buf_v.at[par],
                              gsem.at[par]).wait()
        compute_chunk(par)
        pltpu.async_copy(buf_v.at[par],
                         out_hbm.at[pl.ds(w_base + ci * CHUNK, CHUNK)],
                         osem.at[par])
        return 0

    lax.fori_loop(0, NCHUNK, chunk_body, 0, unroll=False)

    p_last = (NCHUNK - 1) % 2
    pltpu.make_async_copy(
        buf_v.at[1 - p_last],
        out_hbm.at[pl.ds(w_base + (NCHUNK - 2) * CHUNK, CHUNK)],
        osem.at[1 - p_last]).wait()
    pltpu.make_async_copy(
        buf_v.at[p_last],
        out_hbm.at[pl.ds(w_base + (NCHUNK - 1) * CHUNK, CHUNK)],
        osem.at[p_last]).wait()


@jax.jit
def _run(ids_flat, token_table, pos_tiled, gamma, beta):
    mesh = plsc.VectorSubcoreMesh(core_axis_name="c", subcore_axis_name="s")
    kern = functools.partial(
        pl.kernel,
        out_type=jax.ShapeDtypeStruct((ROWS, DIM), jnp.float32),
        mesh=mesh,
        compiler_params=pltpu.CompilerParams(
            needs_layout_passes=False, use_tc_tiling_on_sc=False),
        scratch_types=[
            pltpu.VMEM((2, CHUNK), jnp.int32),
            pltpu.VMEM((2, CHUNK, DIM), jnp.float32),
            pltpu.VMEM((CHUNK, DIM), jnp.float32),
            pltpu.VMEM((2, DIM), jnp.float32),
            pltpu.SemaphoreType.DMA((2,)),
            pltpu.SemaphoreType.DMA((2,)),
        ],
    )(_sc_body)
    return kern(ids_flat, token_table, pos_tiled, gamma, beta)


def kernel(input_ids, token_table, pos_table, gamma, beta):
    ids_flat = input_ids.reshape(-1).astype(jnp.int32)
    pos_tiled = jnp.concatenate([pos_table] * (CHUNK // SEQ), axis=0)
    out = _run(ids_flat, token_table, pos_tiled,
               gamma.astype(jnp.float32), beta.astype(jnp.float32))
    return out.reshape(BATCH, SEQ, DIM)


# speedup vs baseline: 1.9281x; 1.6541x over previous
import functools

import jax
import jax.numpy as jnp
from jax import lax
from jax.experimental import pallas as pl
from jax.experimental.pallas import tpu as pltpu
from jax.experimental.pallas import tpu_sc as plsc

VOCAB = 1000000
SEQ = 200
BATCH = 4096
DIM = 64

NUM_CORES = 2
NUM_SUBCORES = 16
NW = NUM_CORES * NUM_SUBCORES
ROWS = BATCH * SEQ
ROWS_PER_W = ROWS // NW
CHUNK = 2 * SEQ
NCHUNK = ROWS_PER_W // CHUNK
GROUP = 8
NGROUP = CHUNK // GROUP

_Q = DIM // 16


def _sc_body(ids_hbm, table_hbm, pos_hbm, gamma_hbm, beta_hbm, out_hbm,
             idx_v, buf_v, pos_v, gb_v, gsem, osem):
    wid = lax.axis_index("s") * NUM_CORES + lax.axis_index("c")
    w_base = wid * ROWS_PER_W

    pltpu.sync_copy(pos_hbm, pos_v)
    pltpu.sync_copy(gamma_hbm, gb_v.at[0])
    pltpu.sync_copy(beta_hbm, gb_v.at[1])

    pltpu.sync_copy(ids_hbm.at[pl.ds(w_base, CHUNK)], idx_v.at[0])
    pltpu.async_copy(table_hbm.at[idx_v.at[0]], buf_v.at[0], gsem.at[0])

    def compute_chunk(p):
        g = [gb_v[0, pl.ds(16 * q, 16)] for q in range(_Q)]
        b = [gb_v[1, pl.ds(16 * q, 16)] for q in range(_Q)]

        @plsc.parallel_loop(0, CHUNK, 1, unroll=GROUP)
        def _row(r):
            x = [buf_v[p, r, pl.ds(16 * q, 16)]
                 + pos_v[r, pl.ds(16 * q, 16)] for q in range(_Q)]
            s = jnp.sum(x[0] + x[1] + x[2] + x[3])
            ss = jnp.sum(x[0] * x[0] + x[1] * x[1]
                         + x[2] * x[2] + x[3] * x[3])
            mean = s * (1.0 / DIM)
            var = ss * (1.0 / DIM) - mean * mean + 1e-12
            i32 = lax.bitcast_convert_type(var, jnp.int32)
            i32 = jnp.int32(0x5F3759DF) - lax.shift_right_logical(i32, 1)
            y = lax.bitcast_convert_type(i32, jnp.float32)
            half = var * 0.5
            y = y * (1.5 - half * y * y)
            y = y * (1.5 - half * y * y)
            y = y * (1.5 - half * y * y)
            meanv = jnp.broadcast_to(mean, (16,))
            rstdv = jnp.broadcast_to(y, (16,))
            for q in range(_Q):
                buf_v[p, r, pl.ds(16 * q, 16)] = (
                    (x[q] - meanv) * (rstdv * g[q]) + b[q])

    def chunk_body(ci, _):
        par = lax.rem(ci, 2)
        nxt = lax.rem(ci + 1, 2)

        @pl.when(ci + 1 < NCHUNK)
        def _prefetch():
            @pl.when(ci >= 1)
            def _drain_out():
                pltpu.make_async_copy(
                    buf_v.at[nxt],
                    out_hbm.at[pl.ds(w_base + (ci - 1) * CHUNK, CHUNK)],
                    osem.at[nxt]).wait()
            base_n = w_base + (ci + 1) * CHUNK
            pltpu.sync_copy(ids_hbm.at[pl.ds(base_n, CHUNK)], idx_v.at[nxt])
            pltpu.async_copy(table_hbm.at[idx_v.at[nxt]], buf_v.at[nxt],
                             gsem.at[nxt])

        pltpu.make_async_copy(table_hbm.at[idx_v.at[par]], buf_v.at[par],
                              gsem.at[par]).wait()
        compute_chunk(par)
        pltpu.async_copy(buf_v.at[par],
                         out_hbm.at[pl.ds(w_base + ci * CHUNK, CHUNK)],
                         osem.at[par])
        return 0

    lax.fori_loop(0, NCHUNK, chunk_body, 0, unroll=False)

    p_last = (NCHUNK - 1) % 2
    pltpu.make_async_copy(
        buf_v.at[1 - p_last],
        out_hbm.at[pl.ds(w_base + (NCHUNK - 2) * CHUNK, CHUNK)],
        osem.at[1 - p_last]).wait()
    pltpu.make_async_copy(
        buf_v.at[p_last],
        out_hbm.at[pl.ds(w_base + (NCHUNK - 1) * CHUNK, CHUNK)],
        osem.at[p_last]).wait()


@jax.jit
def _run(ids_flat, token_table, pos_tiled, gamma, beta):
    mesh = plsc.VectorSubcoreMesh(core_axis_name="c", subcore_axis_name="s")
    kern = functools.partial(
        pl.kernel,
        out_type=jax.ShapeDtypeStruct((ROWS, DIM), jnp.float32),
        mesh=mesh,
        compiler_params=pltpu.CompilerParams(
            needs_layout_passes=False, use_tc_tiling_on_sc=False),
        scratch_types=[
            pltpu.VMEM((2, CHUNK), jnp.int32),
            pltpu.VMEM((2, CHUNK, DIM), jnp.float32),
            pltpu.VMEM((CHUNK, DIM), jnp.float32),
            pltpu.VMEM((2, DIM), jnp.float32),
            pltpu.SemaphoreType.DMA((2,)),
            pltpu.SemaphoreType.DMA((2,)),
        ],
    )(_sc_body)
    return kern(ids_flat, token_table, pos_tiled, gamma, beta)


def kernel(input_ids, token_table, pos_table, gamma, beta):
    ids_flat = input_ids.reshape(-1).astype(jnp.int32)
    pos_tiled = jnp.concatenate([pos_table] * (CHUNK // SEQ), axis=0)
    out = _run(ids_flat, token_table, pos_tiled,
               gamma.astype(jnp.float32), beta.astype(jnp.float32))
    return out.reshape(BATCH, SEQ, DIM)
